# baseline (speedup 1.0000x reference)
import functools

import jax
import jax.numpy as jnp
from jax import lax
from jax.experimental import pallas as pl
from jax.experimental.pallas import tpu as pltpu
from jax.experimental.pallas import tpu_sc as plsc

D_MODEL = 2048
N_TOK = 8192
TOK_BLOCK = 1024
N_BLOCKS = N_TOK // TOK_BLOCK + 1
T_PAD = N_BLOCKS * TOK_BLOCK
IDX_R, IDX_C = 64, 128
NC, NS = 2, 16
NW = NC * NS
TOK_PER_TILE = N_TOK // NW
SC_CHUNK = 16
N_STEPS = TOK_PER_TILE // SC_CHUNK

_vector_mesh = plsc.VectorSubcoreMesh(core_axis_name="c", subcore_axis_name="s")


def _pos_body(idx_ref, pos_ref, meta_ref):
    idx = idx_ref[...]
    m1 = idx.astype(jnp.float32)
    ii = lax.broadcasted_iota(jnp.int32, (IDX_C, IDX_C), 0)
    jj = lax.broadcasted_iota(jnp.int32, (IDX_C, IDX_C), 1)
    upper = (ii <= jj).astype(jnp.float32)
    row_cum = jnp.dot(m1, upper, preferred_element_type=jnp.float32)
    row_tot = row_cum[:, IDX_C - 1:IDX_C]
    i2 = lax.broadcasted_iota(jnp.int32, (IDX_R, IDX_R), 0)
    j2 = lax.broadcasted_iota(jnp.int32, (IDX_R, IDX_R), 1)
    strict_lower = (j2 < i2).astype(jnp.float32)
    row_off = jnp.dot(strict_lower, row_tot, preferred_element_type=jnp.float32)
    cum1 = row_cum + row_off
    c1 = jnp.sum(m1)
    c0 = float(N_TOK) - c1
    c0_pad = jnp.ceil(c0 / float(TOK_BLOCK)) * float(TOK_BLOCK)
    r = lax.broadcasted_iota(jnp.int32, (IDX_R, IDX_C), 0)
    c = lax.broadcasted_iota(jnp.int32, (IDX_R, IDX_C), 1)
    tpos = (r * IDX_C + c).astype(jnp.float32)
    cum0 = (tpos + 1.0) - cum1
    posf = jnp.where(idx == 0, cum0 - 1.0, c0_pad + cum1 - 1.0)
    pos_ref[...] = posf.astype(jnp.int32)
    bc = lax.broadcasted_iota(jnp.int32, (8, 128), 1).astype(jnp.float32)
    meta_ref[...] = (bc * float(TOK_BLOCK) >= c0_pad).astype(jnp.int32)


_DN = (((1,), (1,)), ((), ()))


def _mm_body(e_ref, x_ref, w0_ref, w1_ref, b0_ref, b1_ref, o_ref):
    e = e_ref[pl.program_id(0)]
    x = x_ref[...].astype(jnp.bfloat16)

    @pl.when(e == 0)
    def _():
        y = lax.dot_general(x, w0_ref[...], _DN,
                            preferred_element_type=jnp.float32)
        o_ref[...] = y + b0_ref[...]

    @pl.when(e != 0)
    def _():
        y = lax.dot_general(x, w1_ref[...], _DN,
                            preferred_element_type=jnp.float32)
        o_ref[...] = y + b1_ref[...]


_SC_SCRATCH = [
    pltpu.VMEM((TOK_PER_TILE,), jnp.int32),
    pltpu.VMEM((SC_CHUNK, D_MODEL), jnp.float32),
    pltpu.VMEM((SC_CHUNK, D_MODEL), jnp.float32),
    pltpu.SemaphoreType.DMA,
    pltpu.SemaphoreType.DMA,
    pltpu.SemaphoreType.DMA,
    pltpu.SemaphoreType.DMA,
]


@functools.partial(
    pl.kernel,
    out_type=jax.ShapeDtypeStruct((T_PAD, D_MODEL), jnp.float32),
    mesh=_vector_mesh,
    scratch_types=_SC_SCRATCH)
def _dispatch(x_hbm, i_hbm, o_hbm, idx_v, buf0, buf1, l0, l1, t0, t1):
    wid = lax.axis_index("s") * NC + lax.axis_index("c")
    base = wid * TOK_PER_TILE
    pltpu.sync_copy(i_hbm.at[pl.ds(base, TOK_PER_TILE)], idx_v)
    bufs, lsem, ssem = (buf0, buf1), (l0, l1), (t0, t1)
    pend_ld = [None, None]
    pend_st = [None, None]
    pend_ld[0] = pltpu.async_copy(
        x_hbm.at[pl.ds(base, SC_CHUNK)], bufs[0], lsem[0])
    for s in range(N_STEPS):
        b, nb = s % 2, (s + 1) % 2
        if s + 1 < N_STEPS:
            if pend_st[nb] is not None:
                pend_st[nb].wait()
            pend_ld[nb] = pltpu.async_copy(
                x_hbm.at[pl.ds(base + (s + 1) * SC_CHUNK, SC_CHUNK)],
                bufs[nb], lsem[nb])
        pend_ld[b].wait()
        pend_st[b] = pltpu.async_copy(
            bufs[b], o_hbm.at[idx_v.at[pl.ds(s * SC_CHUNK, SC_CHUNK)]],
            ssem[b])
    pend_st[0].wait()
    pend_st[1].wait()


@functools.partial(
    pl.kernel,
    out_type=jax.ShapeDtypeStruct((N_TOK, D_MODEL), jnp.float32),
    mesh=_vector_mesh,
    scratch_types=_SC_SCRATCH)
def _unpermute(y_hbm, i_hbm, o_hbm, idx_v, buf0, buf1, l0, l1, t0, t1):
    wid = lax.axis_index("s") * NC + lax.axis_index("c")
    base = wid * TOK_PER_TILE
    pltpu.sync_copy(i_hbm.at[pl.ds(base, TOK_PER_TILE)], idx_v)
    bufs, lsem, ssem = (buf0, buf1), (l0, l1), (t0, t1)
    pend_ld = [None, None]
    pend_st = [None, None]
    pend_ld[0] = pltpu.async_copy(
        y_hbm.at[idx_v.at[pl.ds(0, SC_CHUNK)]], bufs[0], lsem[0])
    for s in range(N_STEPS):
        b, nb = s % 2, (s + 1) % 2
        if s + 1 < N_STEPS:
            if pend_st[nb] is not None:
                pend_st[nb].wait()
            pend_ld[nb] = pltpu.async_copy(
                y_hbm.at[idx_v.at[pl.ds((s + 1) * SC_CHUNK, SC_CHUNK)]],
                bufs[nb], lsem[nb])
        pend_ld[b].wait()
        pend_st[b] = pltpu.async_copy(
            bufs[b], o_hbm.at[pl.ds(base + s * SC_CHUNK, SC_CHUNK)], ssem[b])
    pend_st[0].wait()
    pend_st[1].wait()


@jax.jit
def _run(x2d, idx2d, W0, b0, W1, b1):
    pos2d, meta = pl.pallas_call(
        _pos_body,
        out_shape=(
            jax.ShapeDtypeStruct((IDX_R, IDX_C), jnp.int32),
            jax.ShapeDtypeStruct((8, 128), jnp.int32),
        ),
    )(idx2d)
    pos_flat = pos2d.reshape(N_TOK)
    experts = meta[0, :N_BLOCKS]

    w0b = W0.astype(jnp.bfloat16)
    w1b = W1.astype(jnp.bfloat16)

    x_sorted = _dispatch(x2d, pos_flat)

    y_sorted = pl.pallas_call(
        _mm_body,
        grid_spec=pltpu.PrefetchScalarGridSpec(
            num_scalar_prefetch=1,
            grid=(N_BLOCKS,),
            in_specs=[
                pl.BlockSpec((TOK_BLOCK, D_MODEL), lambda i, e: (i, 0)),
                pl.BlockSpec((D_MODEL, D_MODEL), lambda i, e: (0, 0)),
                pl.BlockSpec((D_MODEL, D_MODEL), lambda i, e: (0, 0)),
                pl.BlockSpec((1, D_MODEL), lambda i, e: (0, 0)),
                pl.BlockSpec((1, D_MODEL), lambda i, e: (0, 0)),
            ],
            out_specs=pl.BlockSpec((TOK_BLOCK, D_MODEL), lambda i, e: (i, 0)),
        ),
        out_shape=jax.ShapeDtypeStruct((T_PAD, D_MODEL), jnp.float32),
        compiler_params=pltpu.CompilerParams(
            dimension_semantics=("arbitrary",)),
    )(experts, x_sorted, w0b, w1b,
      b0.reshape(1, D_MODEL), b1.reshape(1, D_MODEL))

    return _unpermute(y_sorted, pos_flat)


def kernel(hidden_states, multiway_indices, W0, b0, W1, b1):
    batch, seq, d = hidden_states.shape
    x2d = hidden_states.reshape(batch * seq, d)
    idx2d = multiway_indices.astype(jnp.int32).reshape(IDX_R, IDX_C)
    out = _run(x2d, idx2d, W0, b0, W1, b1)
    return out.reshape(batch, seq, d)


# speedup vs baseline: 1.0209x; 1.0209x over previous
import functools

import jax
import jax.numpy as jnp
from jax import lax
from jax.experimental import pallas as pl
from jax.experimental.pallas import tpu as pltpu
from jax.experimental.pallas import tpu_sc as plsc

D_MODEL = 2048
N_TOK = 8192
TOK_BLOCK = 512
N_BLOCKS = N_TOK // TOK_BLOCK + 1
T_PAD = N_BLOCKS * TOK_BLOCK
IDX_R, IDX_C = 64, 128
NC, NS = 2, 16
NW = NC * NS
TOK_PER_TILE = N_TOK // NW
SC_CHUNK = 16
N_STEPS = TOK_PER_TILE // SC_CHUNK

_vector_mesh = plsc.VectorSubcoreMesh(core_axis_name="c", subcore_axis_name="s")


def _pos_body(idx_ref, pos_ref, meta_ref):
    idx = idx_ref[...]
    m1 = idx.astype(jnp.float32)
    ii = lax.broadcasted_iota(jnp.int32, (IDX_C, IDX_C), 0)
    jj = lax.broadcasted_iota(jnp.int32, (IDX_C, IDX_C), 1)
    upper = (ii <= jj).astype(jnp.float32)
    row_cum = jnp.dot(m1, upper, preferred_element_type=jnp.float32)
    row_tot = row_cum[:, IDX_C - 1:IDX_C]
    i2 = lax.broadcasted_iota(jnp.int32, (IDX_R, IDX_R), 0)
    j2 = lax.broadcasted_iota(jnp.int32, (IDX_R, IDX_R), 1)
    strict_lower = (j2 < i2).astype(jnp.float32)
    row_off = jnp.dot(strict_lower, row_tot, preferred_element_type=jnp.float32)
    cum1 = row_cum + row_off
    c1 = jnp.sum(m1)
    c0 = float(N_TOK) - c1
    c0_pad = jnp.ceil(c0 / float(TOK_BLOCK)) * float(TOK_BLOCK)
    r = lax.broadcasted_iota(jnp.int32, (IDX_R, IDX_C), 0)
    c = lax.broadcasted_iota(jnp.int32, (IDX_R, IDX_C), 1)
    tpos = (r * IDX_C + c).astype(jnp.float32)
    cum0 = (tpos + 1.0) - cum1
    posf = jnp.where(idx == 0, cum0 - 1.0, c0_pad + cum1 - 1.0)
    pos_ref[...] = posf.astype(jnp.int32)
    bc = lax.broadcasted_iota(jnp.int32, (8, 128), 1).astype(jnp.float32)
    meta_ref[...] = (bc * float(TOK_BLOCK) >= c0_pad).astype(jnp.int32)


def _mm_body(e_ref, x_ref, w0_ref, w1_ref, b0_ref, b1_ref, o_ref):
    e = e_ref[pl.program_id(0)]
    x = x_ref[...].astype(jnp.bfloat16)

    @pl.when(e == 0)
    def _():
        y = jnp.dot(x, w0_ref[...], preferred_element_type=jnp.float32)
        o_ref[...] = y + b0_ref[...]

    @pl.when(e != 0)
    def _():
        y = jnp.dot(x, w1_ref[...], preferred_element_type=jnp.float32)
        o_ref[...] = y + b1_ref[...]


_SC_SCRATCH = [
    pltpu.VMEM((TOK_PER_TILE,), jnp.int32),
    pltpu.VMEM((SC_CHUNK, D_MODEL), jnp.float32),
    pltpu.VMEM((SC_CHUNK, D_MODEL), jnp.float32),
    pltpu.SemaphoreType.DMA,
    pltpu.SemaphoreType.DMA,
    pltpu.SemaphoreType.DMA,
    pltpu.SemaphoreType.DMA,
]


@functools.partial(
    pl.kernel,
    out_type=jax.ShapeDtypeStruct((T_PAD, D_MODEL), jnp.float32),
    mesh=_vector_mesh,
    scratch_types=_SC_SCRATCH)
def _dispatch(x_hbm, i_hbm, o_hbm, idx_v, buf0, buf1, l0, l1, t0, t1):
    wid = lax.axis_index("s") * NC + lax.axis_index("c")
    base = wid * TOK_PER_TILE
    pltpu.sync_copy(i_hbm.at[pl.ds(base, TOK_PER_TILE)], idx_v)
    bufs, lsem, ssem = (buf0, buf1), (l0, l1), (t0, t1)
    pend_ld = [None, None]
    pend_st = [None, None]
    pend_ld[0] = pltpu.async_copy(
        x_hbm.at[pl.ds(base, SC_CHUNK)], bufs[0], lsem[0])
    for s in range(N_STEPS):
        b, nb = s % 2, (s + 1) % 2
        if s + 1 < N_STEPS:
            if pend_st[nb] is not None:
                pend_st[nb].wait()
            pend_ld[nb] = pltpu.async_copy(
                x_hbm.at[pl.ds(base + (s + 1) * SC_CHUNK, SC_CHUNK)],
                bufs[nb], lsem[nb])
        pend_ld[b].wait()
        pend_st[b] = pltpu.async_copy(
            bufs[b], o_hbm.at[idx_v.at[pl.ds(s * SC_CHUNK, SC_CHUNK)]],
            ssem[b])
    pend_st[0].wait()
    pend_st[1].wait()


@functools.partial(
    pl.kernel,
    out_type=jax.ShapeDtypeStruct((N_TOK, D_MODEL), jnp.float32),
    mesh=_vector_mesh,
    scratch_types=_SC_SCRATCH)
def _unpermute(y_hbm, i_hbm, o_hbm, idx_v, buf0, buf1, l0, l1, t0, t1):
    wid = lax.axis_index("s") * NC + lax.axis_index("c")
    base = wid * TOK_PER_TILE
    pltpu.sync_copy(i_hbm.at[pl.ds(base, TOK_PER_TILE)], idx_v)
    bufs, lsem, ssem = (buf0, buf1), (l0, l1), (t0, t1)
    pend_ld = [None, None]
    pend_st = [None, None]
    pend_ld[0] = pltpu.async_copy(
        y_hbm.at[idx_v.at[pl.ds(0, SC_CHUNK)]], bufs[0], lsem[0])
    for s in range(N_STEPS):
        b, nb = s % 2, (s + 1) % 2
        if s + 1 < N_STEPS:
            if pend_st[nb] is not None:
                pend_st[nb].wait()
            pend_ld[nb] = pltpu.async_copy(
                y_hbm.at[idx_v.at[pl.ds((s + 1) * SC_CHUNK, SC_CHUNK)]],
                bufs[nb], lsem[nb])
        pend_ld[b].wait()
        pend_st[b] = pltpu.async_copy(
            bufs[b], o_hbm.at[pl.ds(base + s * SC_CHUNK, SC_CHUNK)], ssem[b])
    pend_st[0].wait()
    pend_st[1].wait()


@jax.jit
def _run(x2d, idx2d, W0, b0, W1, b1):
    pos2d, meta = pl.pallas_call(
        _pos_body,
        out_shape=(
            jax.ShapeDtypeStruct((IDX_R, IDX_C), jnp.int32),
            jax.ShapeDtypeStruct((8, 128), jnp.int32),
        ),
    )(idx2d)
    pos_flat = pos2d.reshape(N_TOK)
    experts = meta[0, :N_BLOCKS]

    w0b = W0.T.astype(jnp.bfloat16)
    w1b = W1.T.astype(jnp.bfloat16)

    x_sorted = _dispatch(x2d, pos_flat)

    y_sorted = pl.pallas_call(
        _mm_body,
        grid_spec=pltpu.PrefetchScalarGridSpec(
            num_scalar_prefetch=1,
            grid=(N_BLOCKS,),
            in_specs=[
                pl.BlockSpec((TOK_BLOCK, D_MODEL), lambda i, e: (i, 0)),
                pl.BlockSpec((D_MODEL, D_MODEL), lambda i, e: (0, 0)),
                pl.BlockSpec((D_MODEL, D_MODEL), lambda i, e: (0, 0)),
                pl.BlockSpec((1, D_MODEL), lambda i, e: (0, 0)),
                pl.BlockSpec((1, D_MODEL), lambda i, e: (0, 0)),
            ],
            out_specs=pl.BlockSpec((TOK_BLOCK, D_MODEL), lambda i, e: (i, 0)),
        ),
        out_shape=jax.ShapeDtypeStruct((T_PAD, D_MODEL), jnp.float32),
        compiler_params=pltpu.CompilerParams(
            dimension_semantics=("arbitrary",)),
    )(experts, x_sorted, w0b, w1b,
      b0.reshape(1, D_MODEL), b1.reshape(1, D_MODEL))

    return _unpermute(y_sorted, pos_flat)


def kernel(hidden_states, multiway_indices, W0, b0, W1, b1):
    batch, seq, d = hidden_states.shape
    x2d = hidden_states.reshape(batch * seq, d)
    idx2d = multiway_indices.astype(jnp.int32).reshape(IDX_R, IDX_C)
    out = _run(x2d, idx2d, W0, b0, W1, b1)
    return out.reshape(batch, seq, d)


# speedup vs baseline: 1.0270x; 1.0059x over previous
import functools

import jax
import jax.numpy as jnp
from jax import lax
from jax.experimental import pallas as pl
from jax.experimental.pallas import tpu as pltpu
from jax.experimental.pallas import tpu_sc as plsc

D_MODEL = 2048
N_TOK = 8192
TOK_BLOCK = 512
N_BLOCKS = N_TOK // TOK_BLOCK + 1
T_PAD = N_BLOCKS * TOK_BLOCK
IDX_R, IDX_C = 64, 128
NC, NS = 2, 16
NW = NC * NS
TOK_PER_TILE = N_TOK // NW
SC_CHUNK = 16
N_STEPS = TOK_PER_TILE // SC_CHUNK

_vector_mesh = plsc.VectorSubcoreMesh(core_axis_name="c", subcore_axis_name="s")


def _pos_body(idx_ref, pos_ref, meta_ref):
    idx = idx_ref[...]
    m1 = idx.astype(jnp.float32)
    ii = lax.broadcasted_iota(jnp.int32, (IDX_C, IDX_C), 0)
    jj = lax.broadcasted_iota(jnp.int32, (IDX_C, IDX_C), 1)
    upper = (ii <= jj).astype(jnp.float32)
    row_cum = jnp.dot(m1, upper, preferred_element_type=jnp.float32)
    row_tot = row_cum[:, IDX_C - 1:IDX_C]
    i2 = lax.broadcasted_iota(jnp.int32, (IDX_R, IDX_R), 0)
    j2 = lax.broadcasted_iota(jnp.int32, (IDX_R, IDX_R), 1)
    strict_lower = (j2 < i2).astype(jnp.float32)
    row_off = jnp.dot(strict_lower, row_tot, preferred_element_type=jnp.float32)
    cum1 = row_cum + row_off
    c1 = jnp.sum(m1)
    c0 = float(N_TOK) - c1
    c0_pad = jnp.ceil(c0 / float(TOK_BLOCK)) * float(TOK_BLOCK)
    r = lax.broadcasted_iota(jnp.int32, (IDX_R, IDX_C), 0)
    c = lax.broadcasted_iota(jnp.int32, (IDX_R, IDX_C), 1)
    tpos = (r * IDX_C + c).astype(jnp.float32)
    cum0 = (tpos + 1.0) - cum1
    posf = jnp.where(idx == 0, cum0 - 1.0, c0_pad + cum1 - 1.0)
    pos_ref[...] = posf.astype(jnp.int32)
    bc = lax.broadcasted_iota(jnp.int32, (8, 128), 1).astype(jnp.float32)
    meta_ref[...] = (bc * float(TOK_BLOCK) >= c0_pad).astype(jnp.int32)


def _mm_body(e_ref, x_ref, w0_ref, w1_ref, b0_ref, b1_ref, o_ref):
    e = e_ref[pl.program_id(0)]
    x = x_ref[...].astype(jnp.bfloat16)

    @pl.when(e == 0)
    def _():
        y = jnp.dot(x, w0_ref[...], preferred_element_type=jnp.float32)
        o_ref[...] = y + b0_ref[...]

    @pl.when(e != 0)
    def _():
        y = jnp.dot(x, w1_ref[...], preferred_element_type=jnp.float32)
        o_ref[...] = y + b1_ref[...]


_SC_SCRATCH = [
    pltpu.VMEM((TOK_PER_TILE,), jnp.int32),
    pltpu.VMEM((SC_CHUNK, D_MODEL), jnp.float32),
    pltpu.VMEM((SC_CHUNK, D_MODEL), jnp.float32),
    pltpu.VMEM((SC_CHUNK, D_MODEL), jnp.float32),
    pltpu.SemaphoreType.DMA,
    pltpu.SemaphoreType.DMA,
    pltpu.SemaphoreType.DMA,
    pltpu.SemaphoreType.DMA,
    pltpu.SemaphoreType.DMA,
    pltpu.SemaphoreType.DMA,
]
NBUF = 3


@functools.partial(
    pl.kernel,
    out_type=jax.ShapeDtypeStruct((T_PAD, D_MODEL), jnp.float32),
    mesh=_vector_mesh,
    scratch_types=_SC_SCRATCH)
def _dispatch(x_hbm, i_hbm, o_hbm, idx_v, buf0, buf1, buf2,
              l0, l1, l2, t0, t1, t2):
    wid = lax.axis_index("s") * NC + lax.axis_index("c")
    base = wid * TOK_PER_TILE
    pltpu.sync_copy(i_hbm.at[pl.ds(base, TOK_PER_TILE)], idx_v)
    bufs, lsem, ssem = (buf0, buf1, buf2), (l0, l1, l2), (t0, t1, t2)
    pend_ld = [None] * NBUF
    pend_st = [None] * NBUF
    pend_ld[0] = pltpu.async_copy(
        x_hbm.at[pl.ds(base, SC_CHUNK)], bufs[0], lsem[0])
    pend_ld[1] = pltpu.async_copy(
        x_hbm.at[pl.ds(base + SC_CHUNK, SC_CHUNK)], bufs[1], lsem[1])
    for s in range(N_STEPS):
        b, nb = s % NBUF, (s + 2) % NBUF
        if s + 2 < N_STEPS:
            if pend_st[nb] is not None:
                pend_st[nb].wait()
            pend_ld[nb] = pltpu.async_copy(
                x_hbm.at[pl.ds(base + (s + 2) * SC_CHUNK, SC_CHUNK)],
                bufs[nb], lsem[nb])
        pend_ld[b].wait()
        pend_st[b] = pltpu.async_copy(
            bufs[b], o_hbm.at[idx_v.at[pl.ds(s * SC_CHUNK, SC_CHUNK)]],
            ssem[b])
    for p in pend_st:
        if p is not None:
            p.wait()


@functools.partial(
    pl.kernel,
    out_type=jax.ShapeDtypeStruct((N_TOK, D_MODEL), jnp.float32),
    mesh=_vector_mesh,
    scratch_types=_SC_SCRATCH)
def _unpermute(y_hbm, i_hbm, o_hbm, idx_v, buf0, buf1, buf2,
               l0, l1, l2, t0, t1, t2):
    wid = lax.axis_index("s") * NC + lax.axis_index("c")
    base = wid * TOK_PER_TILE
    pltpu.sync_copy(i_hbm.at[pl.ds(base, TOK_PER_TILE)], idx_v)
    bufs, lsem, ssem = (buf0, buf1, buf2), (l0, l1, l2), (t0, t1, t2)
    pend_ld = [None] * NBUF
    pend_st = [None] * NBUF
    pend_ld[0] = pltpu.async_copy(
        y_hbm.at[idx_v.at[pl.ds(0, SC_CHUNK)]], bufs[0], lsem[0])
    pend_ld[1] = pltpu.async_copy(
        y_hbm.at[idx_v.at[pl.ds(SC_CHUNK, SC_CHUNK)]], bufs[1], lsem[1])
    for s in range(N_STEPS):
        b, nb = s % NBUF, (s + 2) % NBUF
        if s + 2 < N_STEPS:
            if pend_st[nb] is not None:
                pend_st[nb].wait()
            pend_ld[nb] = pltpu.async_copy(
                y_hbm.at[idx_v.at[pl.ds((s + 2) * SC_CHUNK, SC_CHUNK)]],
                bufs[nb], lsem[nb])
        pend_ld[b].wait()
        pend_st[b] = pltpu.async_copy(
            bufs[b], o_hbm.at[pl.ds(base + s * SC_CHUNK, SC_CHUNK)], ssem[b])
    for p in pend_st:
        if p is not None:
            p.wait()


@jax.jit
def _run(x2d, idx2d, W0, b0, W1, b1):
    pos2d, meta = pl.pallas_call(
        _pos_body,
        out_shape=(
            jax.ShapeDtypeStruct((IDX_R, IDX_C), jnp.int32),
            jax.ShapeDtypeStruct((8, 128), jnp.int32),
        ),
    )(idx2d)
    pos_flat = pos2d.reshape(N_TOK)
    experts = meta[0, :N_BLOCKS]

    w0b = W0.T.astype(jnp.bfloat16)
    w1b = W1.T.astype(jnp.bfloat16)

    x_sorted = _dispatch(x2d, pos_flat)

    y_sorted = pl.pallas_call(
        _mm_body,
        grid_spec=pltpu.PrefetchScalarGridSpec(
            num_scalar_prefetch=1,
            grid=(N_BLOCKS,),
            in_specs=[
                pl.BlockSpec((TOK_BLOCK, D_MODEL), lambda i, e: (i, 0)),
                pl.BlockSpec((D_MODEL, D_MODEL), lambda i, e: (0, 0)),
                pl.BlockSpec((D_MODEL, D_MODEL), lambda i, e: (0, 0)),
                pl.BlockSpec((1, D_MODEL), lambda i, e: (0, 0)),
                pl.BlockSpec((1, D_MODEL), lambda i, e: (0, 0)),
            ],
            out_specs=pl.BlockSpec((TOK_BLOCK, D_MODEL), lambda i, e: (i, 0)),
        ),
        out_shape=jax.ShapeDtypeStruct((T_PAD, D_MODEL), jnp.float32),
        compiler_params=pltpu.CompilerParams(
            dimension_semantics=("arbitrary",)),
    )(experts, x_sorted, w0b, w1b,
      b0.reshape(1, D_MODEL), b1.reshape(1, D_MODEL))

    return _unpermute(y_sorted, pos_flat)


def kernel(hidden_states, multiway_indices, W0, b0, W1, b1):
    batch, seq, d = hidden_states.shape
    x2d = hidden_states.reshape(batch * seq, d)
    idx2d = multiway_indices.astype(jnp.int32).reshape(IDX_R, IDX_C)
    out = _run(x2d, idx2d, W0, b0, W1, b1)
    return out.reshape(batch, seq, d)
